# baseline (speedup 1.0000x reference)
import jax
import jax.numpy as jnp
from jax import lax
from jax.experimental import pallas as pl
from jax.experimental.pallas import tpu as pltpu
from jax.experimental.pallas import tpu_sc as plsc

N = 10000
D = 128
E = 320000

NC = 2
NS = 16
NW = NC * NS

K = 128
CPW = 80
CH_TOT = NW * CPW
CPD = CH_TOT // NW
E_PAD = CH_TOT * K
N_PAD = 10240
RPT = N_PAD // NS

_MESH = dict(core_axis_name="c", subcore_axis_name="s")


def _deg_body(dst_hbm, zeros_hbm, degs_hbm, idx_v, ones_v, acc):
    c = lax.axis_index("c")
    s = lax.axis_index("s")
    wid = s * NC + c
    pltpu.sync_copy(zeros_hbm.at[pl.ds(s * RPT, RPT)], acc.at[pl.ds(s * RPT, RPT)])
    for i in range(K // 16):
        ones_v[pl.ds(i * 16, 16)] = jnp.ones((16,), jnp.float32)
    pltpu.sync_copy(dst_hbm.at[pl.ds(wid * CPD, CPD)], idx_v)
    plsc.subcore_barrier()

    def chunk(j, carry):
        pltpu.sync_copy(ones_v, acc.at[idx_v.at[j]], add=True)
        return carry

    lax.fori_loop(0, CPD, chunk, 0)
    plsc.subcore_barrier()
    pltpu.sync_copy(acc.at[pl.ds(s * RPT, RPT)], degs_hbm.at[c, pl.ds(s * RPT, RPT)])


def _deg_call(dst3, zeros1):
    return pl.kernel(
        _deg_body,
        out_type=jax.ShapeDtypeStruct((NC, N_PAD), jnp.float32),
        mesh=plsc.VectorSubcoreMesh(**_MESH),
        scratch_types=[
            pltpu.VMEM((CPD, K), jnp.int32),
            pltpu.VMEM((K,), jnp.float32),
            pltpu.VMEM_SHARED((N_PAD,), jnp.float32),
        ],
    )(dst3, zeros1)


def _agg_body(hp_hbm, src_hbm, dst_hbm, zeros_hbm, aggs_hbm,
              sidx, didx, rows_v, acc, sem):
    c = lax.axis_index("c")
    s = lax.axis_index("s")
    wid = s * NC + c
    base = wid * CPW
    pltpu.sync_copy(zeros_hbm.at[pl.ds(s * RPT, RPT)], acc.at[pl.ds(s * RPT, RPT)])
    pltpu.sync_copy(src_hbm.at[pl.ds(base, CPW)], sidx)
    pltpu.sync_copy(dst_hbm.at[pl.ds(base, CPW)], didx)
    plsc.subcore_barrier()

    def chunk(j, carry):
        pltpu.async_copy(hp_hbm.at[sidx.at[j]], rows_v, sem).wait()
        pltpu.sync_copy(rows_v, acc.at[didx.at[j]], add=True)
        return carry

    lax.fori_loop(0, CPW, chunk, 0)
    plsc.subcore_barrier()
    pltpu.sync_copy(acc.at[pl.ds(s * RPT, RPT)], aggs_hbm.at[c, pl.ds(s * RPT, RPT)])


def _agg_call(hp, src3, dst3, zeros2):
    return pl.kernel(
        _agg_body,
        out_type=jax.ShapeDtypeStruct((NC, N_PAD, D), jnp.float32),
        mesh=plsc.VectorSubcoreMesh(**_MESH),
        scratch_types=[
            pltpu.VMEM((CPW, K), jnp.int32),
            pltpu.VMEM((CPW, K), jnp.int32),
            pltpu.VMEM((K, D), jnp.float32),
            pltpu.VMEM_SHARED((N_PAD, D), jnp.float32),
            pltpu.SemaphoreType.DMA,
        ],
    )(hp, src3, dst3, zeros2)


R = 2048
G = N_PAD // R


def _dinv_of(degs_ref):
    deg = degs_ref[:, 0] + degs_ref[:, 1] + 1.0
    return lax.rsqrt(deg)


def _mm_pre_body(x_ref, w_ref, degs_ref, hp_ref):
    dinv = _dinv_of(degs_ref)
    h = jnp.dot(x_ref[...], w_ref[...], preferred_element_type=jnp.float32)
    hp_ref[...] = h * dinv[:, None]


def _mm_pre(x, w, degs):
    return pl.pallas_call(
        _mm_pre_body,
        grid=(G,),
        in_specs=[
            pl.BlockSpec((R, D), lambda i: (i, 0)),
            pl.BlockSpec((D, D), lambda i: (0, 0)),
            pl.BlockSpec((R, NC), lambda i: (i, 0)),
        ],
        out_specs=pl.BlockSpec((R, D), lambda i: (i, 0)),
        out_shape=jax.ShapeDtypeStruct((N_PAD, D), jnp.float32),
    )(x, w, degs)


def _post_mm_body(aggs_ref, hp1_ref, degs_ref, w2_ref, b1_ref, h1_ref, hp2_ref):
    dinv = _dinv_of(degs_ref)
    agg = aggs_ref[0] + aggs_ref[1]
    h1 = jnp.tanh(dinv[:, None] * (agg + hp1_ref[...]) + b1_ref[...][None, :])
    h1_ref[...] = h1
    hp2_ref[...] = jnp.dot(h1, w2_ref[...],
                           preferred_element_type=jnp.float32) * dinv[:, None]


def _post_mm(aggs1, hp1, degs, w2, b1):
    return pl.pallas_call(
        _post_mm_body,
        grid=(G,),
        in_specs=[
            pl.BlockSpec((NC, R, D), lambda i: (0, i, 0)),
            pl.BlockSpec((R, D), lambda i: (i, 0)),
            pl.BlockSpec((R, NC), lambda i: (i, 0)),
            pl.BlockSpec((D, D), lambda i: (0, 0)),
            pl.BlockSpec((D,), lambda i: (0,)),
        ],
        out_specs=[
            pl.BlockSpec((R, D), lambda i: (i, 0)),
            pl.BlockSpec((R, D), lambda i: (i, 0)),
        ],
        out_shape=[
            jax.ShapeDtypeStruct((N_PAD, D), jnp.float32),
            jax.ShapeDtypeStruct((N_PAD, D), jnp.float32),
        ],
    )(aggs1, hp1, degs, w2, b1)


def _post_body(aggs_ref, hp2_ref, degs_ref, b2_ref, h2_ref):
    dinv = _dinv_of(degs_ref)
    agg = aggs_ref[0] + aggs_ref[1]
    h2_ref[...] = jnp.tanh(dinv[:, None] * (agg + hp2_ref[...]) + b2_ref[...][None, :])


def _post(aggs2, hp2, degs, b2):
    return pl.pallas_call(
        _post_body,
        grid=(G,),
        in_specs=[
            pl.BlockSpec((NC, R, D), lambda i: (0, i, 0)),
            pl.BlockSpec((R, D), lambda i: (i, 0)),
            pl.BlockSpec((R, NC), lambda i: (i, 0)),
            pl.BlockSpec((D,), lambda i: (0,)),
        ],
        out_specs=pl.BlockSpec((R, D), lambda i: (i, 0)),
        out_shape=jax.ShapeDtypeStruct((N_PAD, D), jnp.float32),
    )(aggs2, hp2, degs, b2)


def kernel(x, edge_index, W1, b1, W2, b2):
    pad = E_PAD - E
    src3 = jnp.concatenate(
        [edge_index[0], jnp.zeros((pad,), jnp.int32)]).reshape(CH_TOT, K)
    dst3 = jnp.concatenate(
        [edge_index[1], jnp.full((pad,), N_PAD - 1, jnp.int32)]).reshape(CH_TOT, K)
    zeros1 = jnp.zeros((N_PAD,), jnp.float32)
    x_pad = jnp.concatenate([x, jnp.zeros((N_PAD - N, D), jnp.float32)])
    zeros2 = jnp.zeros((N_PAD, D), jnp.float32)

    degs = _deg_call(dst3, zeros1).T
    hp1 = _mm_pre(x_pad, W1, degs)
    aggs1 = _agg_call(hp1, src3, dst3, zeros2)
    h1, hp2 = _post_mm(aggs1, hp1, degs, W2, b1)
    aggs2 = _agg_call(hp2, src3, dst3, zeros2)
    h2 = _post(aggs2, hp2, degs, b2)
    return jnp.stack([h1[:N], h2[:N]], axis=1)


# speedup vs baseline: 1.4544x; 1.4544x over previous
import jax
import jax.numpy as jnp
from jax import lax
from jax.experimental import pallas as pl
from jax.experimental.pallas import tpu as pltpu
from jax.experimental.pallas import tpu_sc as plsc

N = 10000
D = 128
E = 320000

NC = 2
NS = 16
NW = NC * NS

K = 128
CPW = 79
CH_TOT = NW * CPW
CPD = CH_TOT // NW
E_PAD = CH_TOT * K
N_PAD = 10240
RPT = N_PAD // NS

_MESH = dict(core_axis_name="c", subcore_axis_name="s")


def _deg_body(dst_hbm, zeros_hbm, degs_hbm, idx_v, ones_v, acc):
    c = lax.axis_index("c")
    s = lax.axis_index("s")
    wid = s * NC + c
    pltpu.sync_copy(zeros_hbm.at[pl.ds(s * RPT, RPT)], acc.at[pl.ds(s * RPT, RPT)])
    for i in range(K // 16):
        ones_v[pl.ds(i * 16, 16)] = jnp.ones((16,), jnp.float32)
    pltpu.sync_copy(dst_hbm.at[wid], idx_v)
    plsc.subcore_barrier()

    def chunk(j, carry):
        pltpu.sync_copy(ones_v, acc.at[idx_v.at[j]], add=True)
        return carry

    lax.fori_loop(0, CPD, chunk, 0)
    plsc.subcore_barrier()
    pltpu.sync_copy(acc.at[pl.ds(s * RPT, RPT)], degs_hbm.at[c, pl.ds(s * RPT, RPT)])


def _deg_call(dst3, zeros1):
    return pl.kernel(
        _deg_body,
        out_type=jax.ShapeDtypeStruct((NC, N_PAD), jnp.float32),
        mesh=plsc.VectorSubcoreMesh(**_MESH),
        scratch_types=[
            pltpu.VMEM((CPD, K), jnp.int32),
            pltpu.VMEM((K,), jnp.float32),
            pltpu.VMEM_SHARED((N_PAD,), jnp.float32),
        ],
    )(dst3, zeros1)


def _agg_body(hp_hbm, src_hbm, dst_hbm, zeros_hbm, aggs_hbm,
              sidx, didx, rows_v, acc, sem):
    c = lax.axis_index("c")
    s = lax.axis_index("s")
    wid = s * NC + c
    pltpu.sync_copy(zeros_hbm.at[pl.ds(s * RPT, RPT)], acc.at[pl.ds(s * RPT, RPT)])
    pltpu.sync_copy(src_hbm.at[wid], sidx)
    pltpu.sync_copy(dst_hbm.at[wid], didx)
    plsc.subcore_barrier()

    def chunk(j, carry):
        pltpu.async_copy(hp_hbm.at[sidx.at[j]], rows_v, sem).wait()
        pltpu.sync_copy(rows_v, acc.at[didx.at[j]], add=True)
        return carry

    lax.fori_loop(0, CPW, chunk, 0)
    plsc.subcore_barrier()
    pltpu.sync_copy(acc.at[pl.ds(s * RPT, RPT)], aggs_hbm.at[c, pl.ds(s * RPT, RPT)])


def _agg_call(hp, src3, dst3, zeros2):
    return pl.kernel(
        _agg_body,
        out_type=jax.ShapeDtypeStruct((NC, N_PAD, D), jnp.float32),
        mesh=plsc.VectorSubcoreMesh(**_MESH),
        scratch_types=[
            pltpu.VMEM((CPW, K), jnp.int32),
            pltpu.VMEM((CPW, K), jnp.int32),
            pltpu.VMEM((K, D), jnp.float32),
            pltpu.VMEM_SHARED((N_PAD, D), jnp.float32),
            pltpu.SemaphoreType.DMA,
        ],
    )(hp, src3, dst3, zeros2)


R = 2000
G = N // R


def _dinv_of(degs_ref):
    deg = degs_ref[:, 0] + degs_ref[:, 1] + 1.0
    return lax.rsqrt(deg)


def _mm_pre_body(x_ref, w_ref, degs_ref, hp_ref):
    dinv = _dinv_of(degs_ref)
    h = jnp.dot(x_ref[...], w_ref[...], preferred_element_type=jnp.float32)
    hp_ref[...] = h * dinv[:, None]


def _mm_pre(x, w, degs):
    return pl.pallas_call(
        _mm_pre_body,
        grid=(G,),
        in_specs=[
            pl.BlockSpec((R, D), lambda i: (i, 0)),
            pl.BlockSpec((D, D), lambda i: (0, 0)),
            pl.BlockSpec((R, NC), lambda i: (i, 0)),
        ],
        out_specs=pl.BlockSpec((R, D), lambda i: (i, 0)),
        out_shape=jax.ShapeDtypeStruct((N, D), jnp.float32),
    )(x, w, degs)


def _post_mm_body(aggs_ref, hp1_ref, degs_ref, w2_ref, b1_ref, h1_ref, hp2_ref):
    dinv = _dinv_of(degs_ref)
    agg = aggs_ref[0] + aggs_ref[1]
    h1 = jnp.tanh(dinv[:, None] * (agg + hp1_ref[...]) + b1_ref[...][None, :])
    h1_ref[...] = h1
    hp2_ref[...] = jnp.dot(h1, w2_ref[...],
                           preferred_element_type=jnp.float32) * dinv[:, None]


def _post_mm(aggs1, hp1, degs, w2, b1):
    return pl.pallas_call(
        _post_mm_body,
        grid=(G,),
        in_specs=[
            pl.BlockSpec((NC, R, D), lambda i: (0, i, 0)),
            pl.BlockSpec((R, D), lambda i: (i, 0)),
            pl.BlockSpec((R, NC), lambda i: (i, 0)),
            pl.BlockSpec((D, D), lambda i: (0, 0)),
            pl.BlockSpec((D,), lambda i: (0,)),
        ],
        out_specs=[
            pl.BlockSpec((R, D), lambda i: (i, 0)),
            pl.BlockSpec((R, D), lambda i: (i, 0)),
        ],
        out_shape=[
            jax.ShapeDtypeStruct((N, D), jnp.float32),
            jax.ShapeDtypeStruct((N, D), jnp.float32),
        ],
    )(aggs1, hp1, degs, w2, b1)


def _post_body(aggs_ref, hp2_ref, degs_ref, b2_ref, h2_ref):
    dinv = _dinv_of(degs_ref)
    agg = aggs_ref[0] + aggs_ref[1]
    h2_ref[...] = jnp.tanh(dinv[:, None] * (agg + hp2_ref[...]) + b2_ref[...][None, :])


def _post(aggs2, hp2, degs, b2):
    return pl.pallas_call(
        _post_body,
        grid=(G,),
        in_specs=[
            pl.BlockSpec((NC, R, D), lambda i: (0, i, 0)),
            pl.BlockSpec((R, D), lambda i: (i, 0)),
            pl.BlockSpec((R, NC), lambda i: (i, 0)),
            pl.BlockSpec((D,), lambda i: (0,)),
        ],
        out_specs=pl.BlockSpec((R, D), lambda i: (i, 0)),
        out_shape=jax.ShapeDtypeStruct((N, D), jnp.float32),
    )(aggs2, hp2, degs, b2)


def kernel(x, edge_index, W1, b1, W2, b2):
    pad = E_PAD - E
    src3 = jnp.concatenate(
        [edge_index[0], jnp.zeros((pad,), jnp.int32)]).reshape(NW, CPW, K)
    dst3 = jnp.concatenate(
        [edge_index[1], jnp.full((pad,), N_PAD - 1, jnp.int32)]).reshape(NW, CPW, K)
    zeros1 = jnp.zeros((N_PAD,), jnp.float32)
    zeros2 = jnp.zeros((N_PAD, D), jnp.float32)

    degs = _deg_call(dst3, zeros1).T
    hp1 = _mm_pre(x, W1, degs)
    aggs1 = _agg_call(hp1, src3, dst3, zeros2)
    h1, hp2 = _post_mm(aggs1, hp1, degs, W2, b1)
    aggs2 = _agg_call(hp2, src3, dst3, zeros2)
    h2 = _post(aggs2, hp2, degs, b2)
    return jnp.stack([h1, h2], axis=1)


# speedup vs baseline: 1.4780x; 1.0162x over previous
import jax
import jax.numpy as jnp
from jax import lax
from jax.experimental import pallas as pl
from jax.experimental.pallas import tpu as pltpu
from jax.experimental.pallas import tpu_sc as plsc

N = 10000
D = 128
E = 320000

NC = 2
NS = 16
NW = NC * NS

K = 128
CPW = 79
CH_TOT = NW * CPW
CPD = CH_TOT // NW
E_PAD = CH_TOT * K
N_PAD = 10240
RPT = N_PAD // NS

_MESH = dict(core_axis_name="c", subcore_axis_name="s")


def _deg_body(dst_hbm, zeros_hbm, ones_hbm, degs_hbm, idx_v, ones_v, acc):
    c = lax.axis_index("c")
    s = lax.axis_index("s")
    wid = s * NC + c
    pltpu.sync_copy(zeros_hbm.at[pl.ds(s * RPT, RPT)], acc.at[pl.ds(s * RPT, RPT)])
    pltpu.sync_copy(ones_hbm, ones_v)
    pltpu.sync_copy(dst_hbm.at[wid], idx_v)
    plsc.subcore_barrier()

    def chunk(j, carry):
        pltpu.sync_copy(ones_v, acc.at[idx_v.at[j]], add=True)
        return carry

    lax.fori_loop(0, CPD, chunk, 0)
    plsc.subcore_barrier()
    pltpu.sync_copy(acc.at[pl.ds(s * RPT, RPT)], degs_hbm.at[c, pl.ds(s * RPT, RPT)])


def _deg_call(dst3, zeros1, ones1):
    return pl.kernel(
        _deg_body,
        out_type=jax.ShapeDtypeStruct((NC, N_PAD), jnp.float32),
        mesh=plsc.VectorSubcoreMesh(**_MESH),
        scratch_types=[
            pltpu.VMEM((CPD, K), jnp.int32),
            pltpu.VMEM((K,), jnp.float32),
            pltpu.VMEM_SHARED((N_PAD,), jnp.float32),
        ],
    )(dst3, zeros1, ones1)


def _agg_body(hp_hbm, src_hbm, dst_hbm, zeros_hbm, aggs_hbm,
              sidx, didx, rows_v, acc, sem):
    c = lax.axis_index("c")
    s = lax.axis_index("s")
    wid = s * NC + c
    pltpu.sync_copy(zeros_hbm.at[pl.ds(s * RPT, RPT)], acc.at[pl.ds(s * RPT, RPT)])
    pltpu.sync_copy(src_hbm.at[wid], sidx)
    pltpu.sync_copy(dst_hbm.at[wid], didx)
    plsc.subcore_barrier()

    def chunk(j, carry):
        pltpu.async_copy(hp_hbm.at[sidx.at[j]], rows_v, sem).wait()
        pltpu.sync_copy(rows_v, acc.at[didx.at[j]], add=True)
        return carry

    lax.fori_loop(0, CPW, chunk, 0)
    plsc.subcore_barrier()
    pltpu.sync_copy(acc.at[pl.ds(s * RPT, RPT)], aggs_hbm.at[c, pl.ds(s * RPT, RPT)])


def _agg_call(hp, src3, dst3, zeros2):
    return pl.kernel(
        _agg_body,
        out_type=jax.ShapeDtypeStruct((NC, N_PAD, D), jnp.float32),
        mesh=plsc.VectorSubcoreMesh(**_MESH),
        scratch_types=[
            pltpu.VMEM((CPW, K), jnp.int32),
            pltpu.VMEM((CPW, K), jnp.int32),
            pltpu.VMEM((K, D), jnp.float32),
            pltpu.VMEM_SHARED((N_PAD, D), jnp.float32),
            pltpu.SemaphoreType.DMA,
        ],
    )(hp, src3, dst3, zeros2)


R = 2000
G = N // R


def _dinv_of(degs_ref):
    deg = degs_ref[:, 0] + degs_ref[:, 1] + 1.0
    return lax.rsqrt(deg)


def _mm_pre_body(x_ref, w_ref, degs_ref, hp_ref):
    dinv = _dinv_of(degs_ref)
    h = jnp.dot(x_ref[...], w_ref[...], preferred_element_type=jnp.float32)
    hp_ref[...] = h * dinv[:, None]


def _mm_pre(x, w, degs):
    return pl.pallas_call(
        _mm_pre_body,
        grid=(G,),
        in_specs=[
            pl.BlockSpec((R, D), lambda i: (i, 0)),
            pl.BlockSpec((D, D), lambda i: (0, 0)),
            pl.BlockSpec((R, NC), lambda i: (i, 0)),
        ],
        out_specs=pl.BlockSpec((R, D), lambda i: (i, 0)),
        out_shape=jax.ShapeDtypeStruct((N, D), jnp.float32),
    )(x, w, degs)


def _post_mm_body(aggs_ref, hp1_ref, degs_ref, w2_ref, b1_ref, h1_ref, hp2_ref):
    dinv = _dinv_of(degs_ref)
    agg = aggs_ref[0] + aggs_ref[1]
    h1 = jnp.tanh(dinv[:, None] * (agg + hp1_ref[...]) + b1_ref[...][None, :])
    h1_ref[...] = h1
    hp2_ref[...] = jnp.dot(h1, w2_ref[...],
                           preferred_element_type=jnp.float32) * dinv[:, None]


def _post_mm(aggs1, hp1, degs, w2, b1):
    return pl.pallas_call(
        _post_mm_body,
        grid=(G,),
        in_specs=[
            pl.BlockSpec((NC, R, D), lambda i: (0, i, 0)),
            pl.BlockSpec((R, D), lambda i: (i, 0)),
            pl.BlockSpec((R, NC), lambda i: (i, 0)),
            pl.BlockSpec((D, D), lambda i: (0, 0)),
            pl.BlockSpec((D,), lambda i: (0,)),
        ],
        out_specs=[
            pl.BlockSpec((R, D), lambda i: (i, 0)),
            pl.BlockSpec((R, D), lambda i: (i, 0)),
        ],
        out_shape=[
            jax.ShapeDtypeStruct((N, D), jnp.float32),
            jax.ShapeDtypeStruct((N, D), jnp.float32),
        ],
    )(aggs1, hp1, degs, w2, b1)


def _post_body(aggs_ref, hp2_ref, degs_ref, b2_ref, h2_ref):
    dinv = _dinv_of(degs_ref)
    agg = aggs_ref[0] + aggs_ref[1]
    h2_ref[...] = jnp.tanh(dinv[:, None] * (agg + hp2_ref[...]) + b2_ref[...][None, :])


def _post(aggs2, hp2, degs, b2):
    return pl.pallas_call(
        _post_body,
        grid=(G,),
        in_specs=[
            pl.BlockSpec((NC, R, D), lambda i: (0, i, 0)),
            pl.BlockSpec((R, D), lambda i: (i, 0)),
            pl.BlockSpec((R, NC), lambda i: (i, 0)),
            pl.BlockSpec((D,), lambda i: (0,)),
        ],
        out_specs=pl.BlockSpec((R, D), lambda i: (i, 0)),
        out_shape=jax.ShapeDtypeStruct((N, D), jnp.float32),
    )(aggs2, hp2, degs, b2)


def kernel(x, edge_index, W1, b1, W2, b2):
    pad = E_PAD - E
    src3 = jnp.concatenate(
        [edge_index[0], jnp.zeros((pad,), jnp.int32)]).reshape(NW, CPW, K)
    dst3 = jnp.concatenate(
        [edge_index[1], jnp.full((pad,), N_PAD - 1, jnp.int32)]).reshape(NW, CPW, K)
    zeros1 = jnp.zeros((N_PAD,), jnp.float32)
    ones1 = jnp.ones((K,), jnp.float32)
    zeros2 = jnp.zeros((N_PAD, D), jnp.float32)

    degs = _deg_call(dst3, zeros1, ones1).T
    hp1 = _mm_pre(x, W1, degs)
    aggs1 = _agg_call(hp1, src3, dst3, zeros2)
    h1, hp2 = _post_mm(aggs1, hp1, degs, W2, b1)
    aggs2 = _agg_call(hp2, src3, dst3, zeros2)
    h2 = _post(aggs2, hp2, degs, b2)
    return jnp.stack([h1, h2], axis=1)
